# Initial kernel scaffold; baseline (speedup 1.0000x reference)
import jax
import jax.numpy as jnp
from jax.experimental import pallas as pl


def kernel(x, lengths, mask, emb, ln_scale, ln_bias, dwk, W1, b1, W2, b2):
    raise NotImplementedError("write your pallas kernel here")


# speedup vs baseline: 1.9558x; 1.9558x over previous
import jax
import jax.numpy as jnp
from jax.experimental import pallas as pl
from jax.experimental.pallas import tpu as pltpu
from jax.experimental.pallas import tpu_sc as plsc

_DIM = 128
_LAYERS = 4
_FF = 512
_B = 1024
_T = 200
_BB = 16
_GW = 128


def _sc_gather(emb, idx_flat):
    n = idx_flat.shape[1]
    mesh = plsc.VectorSubcoreMesh(core_axis_name="core", subcore_axis_name="subcore")

    @pl.kernel(out_type=jax.ShapeDtypeStruct((n, _DIM), emb.dtype), mesh=mesh)
    def gather_kernel(x_hbm, i_hbm, o_hbm):
        def body(i_vmem, o_vmem):
            pltpu.sync_copy(x_hbm.at[i_vmem.at[0]], o_vmem)

        pltpu.emit_pipeline(
            body,
            grid=(n // _GW,),
            in_specs=[pl.BlockSpec((1, _GW), index_map=lambda i: (0, i))],
            out_specs=[pl.BlockSpec((_GW, _DIM), index_map=lambda i: (i, 0))],
            core_axis_name=("core", "subcore"),
            dimension_semantics=(pltpu.PARALLEL,),
        )(i_hbm, o_hbm)

    return gather_kernel(emb, idx_flat)


def _backbone_body(hg_ref, m_ref, lns_ref, lnb_ref, dwk_ref,
                   w1_ref, b1_ref, w2_ref, b2_ref, o_ref):
    m = m_ref[...]
    h = hg_ref[...] * m
    for i in range(_LAYERS):
        mu = jnp.mean(h, axis=-1, keepdims=True)
        var = jnp.mean((h - mu) ** 2, axis=-1, keepdims=True)
        r = (h - mu) * jax.lax.rsqrt(var + 1e-5) * lns_ref[i] + lnb_ref[i]
        r = r * m
        z = jnp.zeros((_BB, 1, _DIM), jnp.float32)
        prev = jnp.concatenate([z, r[:, :-1, :]], axis=1)
        nxt = jnp.concatenate([r[:, 1:, :], z], axis=1)
        conv = prev * dwk_ref[i, 0] + r * dwk_ref[i, 1] + nxt * dwk_ref[i, 2]
        c2 = conv.reshape(_BB * _T, _DIM)
        f = jax.nn.gelu(
            jnp.dot(c2, w1_ref[i], preferred_element_type=jnp.float32) + b1_ref[i])
        f = jnp.dot(f, w2_ref[i], preferred_element_type=jnp.float32) + b2_ref[i]
        h = (h + f.reshape(_BB, _T, _DIM)) * m
    o_ref[...] = h


def _run_backbone(hg, mask3, ln_scale, ln_bias, dwk, W1, b1, W2, b2,
                  interpret=False):
    grid = (_B // _BB,)
    const = lambda *dims: pl.BlockSpec(dims, lambda i: (0,) * len(dims))
    return pl.pallas_call(
        _backbone_body,
        grid=grid,
        in_specs=[
            pl.BlockSpec((_BB, _T, _DIM), lambda i: (i, 0, 0)),
            pl.BlockSpec((_BB, _T, 1), lambda i: (i, 0, 0)),
            const(_LAYERS, _DIM),
            const(_LAYERS, _DIM),
            const(_LAYERS, 3, _DIM),
            const(_LAYERS, _DIM, _FF),
            const(_LAYERS, _FF),
            const(_LAYERS, _FF, _DIM),
            const(_LAYERS, _DIM),
        ],
        out_specs=pl.BlockSpec((_BB, _T, _DIM), lambda i: (i, 0, 0)),
        out_shape=jax.ShapeDtypeStruct((_B, _T, _DIM), jnp.float32),
        compiler_params=pltpu.CompilerParams(
            dimension_semantics=("parallel",)),
        interpret=interpret,
    )(hg, mask3, ln_scale, ln_bias, dwk, W1, b1, W2, b2)


def kernel(x, lengths, mask, emb, ln_scale, ln_bias, dwk, W1, b1, W2, b2):
    idx_flat = x.reshape(1, _B * _T)
    hg = _sc_gather(emb, idx_flat).reshape(_B, _T, _DIM)
    mask3 = mask[:, :, None]
    return _run_backbone(hg, mask3, ln_scale, ln_bias, dwk, W1, b1, W2, b2)
